# baseline (speedup 1.0000x reference)
import jax
import jax.numpy as jnp
from jax import lax
from jax.experimental import pallas as pl
from jax.experimental.pallas import tpu as pltpu, tpu_sc as plsc

_NW = 32
_TOT = 96 * 512 * 512
_COPY = 80 * 512 * 512
_MASKN = 16 * 512 * 512
_CPW = _COPY // _NW
_MPW = _MASKN // _NW


def _sc_body(x_hbm, m_hbm, o_hbm, sem, sem2):
    c = lax.axis_index("c")
    s = lax.axis_index("s")
    wid = s * 2 + c
    base = wid * _CPW
    cp = pltpu.make_async_copy(
        x_hbm.at[pl.ds(base, _CPW)], o_hbm.at[pl.ds(base, _CPW)], sem
    )
    cp.start()
    mbase = wid * _MPW
    cp2 = pltpu.make_async_copy(
        m_hbm.at[pl.ds(mbase, _MPW)], o_hbm.at[pl.ds(_COPY + mbase, _MPW)], sem2
    )
    cp2.start()
    cp.wait()
    cp2.wait()


def kernel(x, mask):
    xf = x.reshape(_TOT)
    mf = mask.reshape(_MASKN)
    mesh = plsc.VectorSubcoreMesh(core_axis_name="c", subcore_axis_name="s")
    out = pl.kernel(
        _sc_body,
        mesh=mesh,
        out_type=jax.ShapeDtypeStruct((_TOT,), jnp.float32),
        scratch_types=[
            pltpu.SemaphoreType.DMA,
            pltpu.SemaphoreType.DMA,
        ],
    )(xf, mf)
    return out.reshape(x.shape)


# speedup vs baseline: 11.0287x; 11.0287x over previous
import jax
import jax.numpy as jnp
from jax import lax
from jax.experimental import pallas as pl
from jax.experimental.pallas import tpu as pltpu, tpu_sc as plsc

_NW = 32
_TOT = 96 * 512 * 512
_COPY = 80 * 512 * 512
_MASKN = 16 * 512 * 512
_CPW = _COPY // _NW
_MPW = _MASKN // _NW
_VCH = 32768
_NCC = _CPW // _VCH
_NMC = _MPW // _VCH
_NB = 3


def _sigmoid_inplace(buf):
    def body(i, carry):
        sl = pl.ds(i * 16, 16)
        v = buf[sl]
        buf[sl] = 1.0 / (1.0 + jnp.exp(-v))
        return carry

    lax.fori_loop(0, _VCH // 16, body, 0, unroll=8)


def _sc_body(x_hbm, m_hbm, o_hbm, b0, b1, b2, i0, i1, i2, o0, o1, o2):
    bufs = (b0, b1, b2)
    sin = (i0, i1, i2)
    sout = (o0, o1, o2)
    c = lax.axis_index("c")
    s = lax.axis_index("s")
    wid = s * 2 + c
    cbase = wid * _CPW
    mbase = wid * _MPW
    jobs = [(x_hbm, cbase + k * _VCH, cbase + k * _VCH, False) for k in range(_NCC)]
    jobs += [
        (m_hbm, mbase + k * _VCH, _COPY + mbase + k * _VCH, True)
        for k in range(_NMC)
    ]
    n = len(jobs)
    for b in range(_NB):
        src, soff, _, _ = jobs[b]
        pltpu.make_async_copy(src.at[pl.ds(soff, _VCH)], bufs[b], sin[b]).start()
    for k in range(n):
        b = k % _NB
        src, soff, doff, comp = jobs[k]
        pltpu.make_async_copy(src.at[pl.ds(soff, _VCH)], bufs[b], sin[b]).wait()
        if comp:
            _sigmoid_inplace(bufs[b])
        out_cp = pltpu.make_async_copy(bufs[b], o_hbm.at[pl.ds(doff, _VCH)], sout[b])
        out_cp.start()
        if k + _NB < n:
            nsrc, nsoff, _, _ = jobs[k + _NB]
            out_cp.wait()
            pltpu.make_async_copy(
                nsrc.at[pl.ds(nsoff, _VCH)], bufs[b], sin[b]
            ).start()
    for k in range(max(0, n - _NB), n):
        b = k % _NB
        _, _, doff, _ = jobs[k]
        pltpu.make_async_copy(bufs[b], o_hbm.at[pl.ds(doff, _VCH)], sout[b]).wait()


def kernel(x, mask):
    xf = x.reshape(_TOT)
    mf = mask.reshape(_MASKN)
    mesh = plsc.VectorSubcoreMesh(core_axis_name="c", subcore_axis_name="s")
    out = pl.kernel(
        _sc_body,
        mesh=mesh,
        out_type=jax.ShapeDtypeStruct((_TOT,), jnp.float32),
        scratch_types=[
            pltpu.VMEM((_VCH,), jnp.float32),
            pltpu.VMEM((_VCH,), jnp.float32),
            pltpu.VMEM((_VCH,), jnp.float32),
            pltpu.SemaphoreType.DMA,
            pltpu.SemaphoreType.DMA,
            pltpu.SemaphoreType.DMA,
            pltpu.SemaphoreType.DMA,
            pltpu.SemaphoreType.DMA,
            pltpu.SemaphoreType.DMA,
        ],
    )(xf, mf)
    return out.reshape(x.shape)


# speedup vs baseline: 45.2695x; 4.1047x over previous
import jax
import jax.numpy as jnp
from jax.experimental import pallas as pl

_CB = 2
_NCOPY = 40
_NGRID = 48


def _body(x_ref, m_ref, o_ref):
    c = pl.program_id(0)

    @pl.when(c < _NCOPY)
    def _copy():
        o_ref[...] = x_ref[...]

    @pl.when(c >= _NCOPY)
    def _sig():
        o_ref[...] = jax.nn.sigmoid(m_ref[...])


def kernel(x, mask):
    C, H, W = x.shape
    return pl.pallas_call(
        _body,
        grid=(_NGRID,),
        in_specs=[
            pl.BlockSpec((_CB, H, W), lambda c: (jnp.minimum(c, _NCOPY - 1), 0, 0)),
            pl.BlockSpec((_CB, H, W), lambda c: (jnp.maximum(c - _NCOPY, 0), 0, 0)),
        ],
        out_specs=pl.BlockSpec((_CB, H, W), lambda c: (c, 0, 0)),
        out_shape=jax.ShapeDtypeStruct((C, H, W), x.dtype),
    )(x, mask)


# speedup vs baseline: 49.6779x; 1.0974x over previous
import jax
import jax.numpy as jnp
from jax.experimental import pallas as pl

_CB = 4
_NCOPY = 20
_NGRID = 24


def _body(x_ref, m_ref, o_ref):
    c = pl.program_id(0)

    @pl.when(c < _NCOPY)
    def _copy():
        o_ref[...] = x_ref[...]

    @pl.when(c >= _NCOPY)
    def _sig():
        o_ref[...] = jax.nn.sigmoid(m_ref[...])


def kernel(x, mask):
    C, H, W = x.shape
    return pl.pallas_call(
        _body,
        grid=(_NGRID,),
        in_specs=[
            pl.BlockSpec((_CB, H, W), lambda c: (jnp.minimum(c, _NCOPY - 1), 0, 0)),
            pl.BlockSpec((_CB, H, W), lambda c: (jnp.maximum(c - _NCOPY, 0), 0, 0)),
        ],
        out_specs=pl.BlockSpec((_CB, H, W), lambda c: (c, 0, 0)),
        out_shape=jax.ShapeDtypeStruct((C, H, W), x.dtype),
    )(x, mask)
